# baseline (speedup 1.0000x reference)
import jax
import jax.numpy as jnp
from jax.experimental import pallas as pl
from jax.experimental.pallas import tpu as pltpu

_K = 1000
_M = 16384
_N = 128
_CH = 512
_NCH = _M // _CH
_NBUF = 3


def _mm_stream(xt_hbm, w_ref, b_ref, o_hbm, xbuf, obuf, insem, outsem):
    def in_copy(c, slot):
        return pltpu.make_async_copy(
            xt_hbm.at[:, pl.ds(c * _CH, _CH)], xbuf.at[slot], insem.at[slot]
        )

    def out_copy(c, slot):
        return pltpu.make_async_copy(
            obuf.at[slot], o_hbm.at[pl.ds(c * _CH, _CH), :], outsem.at[slot]
        )

    for c in range(_NBUF - 1):
        in_copy(c, c % _NBUF).start()
    for c in range(_NCH):
        nxt = c + _NBUF - 1
        if nxt < _NCH:
            in_copy(nxt, nxt % _NBUF).start()
        in_copy(c, c % _NBUF).wait()
        if c >= 2:
            out_copy(c - 2, c % 2).wait()
        acc = jax.lax.dot_general(
            xbuf[c % _NBUF], w_ref[...], (((0,), (0,)), ((), ())),
            preferred_element_type=jnp.float32,
        )
        obuf[c % 2] = jnp.maximum(acc + b_ref[...], 0.0)
        out_copy(c, c % 2).start()
    for c in range(_NCH - 2, _NCH):
        out_copy(c, c % 2).wait()


@jax.jit
def _run(inputs, weights, bias2d):
    m, k = inputs.shape
    n = weights.shape[1]
    xt = inputs.T
    return pl.pallas_call(
        _mm_stream,
        in_specs=[
            pl.BlockSpec(memory_space=pltpu.MemorySpace.HBM),
            pl.BlockSpec(memory_space=pltpu.MemorySpace.VMEM),
            pl.BlockSpec(memory_space=pltpu.MemorySpace.VMEM),
        ],
        out_specs=pl.BlockSpec(memory_space=pltpu.MemorySpace.HBM),
        out_shape=jax.ShapeDtypeStruct((m, n), jnp.float32),
        scratch_shapes=[
            pltpu.VMEM((_NBUF, _K, _CH), jnp.float32),
            pltpu.VMEM((2, _CH, _N), jnp.float32),
            pltpu.SemaphoreType.DMA((_NBUF,)),
            pltpu.SemaphoreType.DMA((2,)),
        ],
    )(xt, weights, bias2d)


def kernel(inputs, kernel, bias):
    return _run(inputs, kernel, bias.reshape(1, -1))


# speedup vs baseline: 1.0654x; 1.0654x over previous
import jax
import jax.numpy as jnp
from jax.experimental import pallas as pl
from jax.experimental.pallas import tpu as pltpu

_K = 1000
_M = 16384
_N = 128
_CH = 2048
_NCH = _M // _CH
_NBUF = 3


def _mm_stream(xt_hbm, w_ref, b_ref, o_hbm, xbuf, obuf, insem, outsem):
    def in_copy(c, slot):
        return pltpu.make_async_copy(
            xt_hbm.at[:, pl.ds(c * _CH, _CH)], xbuf.at[slot], insem.at[slot]
        )

    def out_copy(c, slot):
        return pltpu.make_async_copy(
            obuf.at[slot], o_hbm.at[pl.ds(c * _CH, _CH), :], outsem.at[slot]
        )

    for c in range(_NBUF - 1):
        in_copy(c, c % _NBUF).start()
    for c in range(_NCH):
        nxt = c + _NBUF - 1
        if nxt < _NCH:
            in_copy(nxt, nxt % _NBUF).start()
        in_copy(c, c % _NBUF).wait()
        if c >= 2:
            out_copy(c - 2, c % 2).wait()
        acc = jax.lax.dot_general(
            xbuf[c % _NBUF], w_ref[...], (((0,), (0,)), ((), ())),
            preferred_element_type=jnp.float32,
        )
        obuf[c % 2] = jnp.maximum(acc + b_ref[...], 0.0)
        out_copy(c, c % 2).start()
    for c in range(_NCH - 2, _NCH):
        out_copy(c, c % 2).wait()


@jax.jit
def _run(inputs, weights, bias2d):
    m, k = inputs.shape
    n = weights.shape[1]
    xt = inputs.T
    return pl.pallas_call(
        _mm_stream,
        in_specs=[
            pl.BlockSpec(memory_space=pltpu.MemorySpace.HBM),
            pl.BlockSpec(memory_space=pltpu.MemorySpace.VMEM),
            pl.BlockSpec(memory_space=pltpu.MemorySpace.VMEM),
        ],
        out_specs=pl.BlockSpec(memory_space=pltpu.MemorySpace.HBM),
        out_shape=jax.ShapeDtypeStruct((m, n), jnp.float32),
        scratch_shapes=[
            pltpu.VMEM((_NBUF, _K, _CH), jnp.float32),
            pltpu.VMEM((2, _CH, _N), jnp.float32),
            pltpu.SemaphoreType.DMA((_NBUF,)),
            pltpu.SemaphoreType.DMA((2,)),
        ],
    )(xt, weights, bias2d)


def kernel(inputs, kernel, bias):
    return _run(inputs, kernel, bias.reshape(1, -1))
